# baseline (speedup 1.0000x reference)
import functools

import jax
import jax.numpy as jnp
from jax.experimental import pallas as pl

_NUM_TOKENS = 200
_TOKEN_SIZE = 64
_BB = 128


def _body(tok_ref, en_ref, w_ref, eb_ref, pe_ref, out_ref):
    e = jnp.dot(en_ref[:], w_ref[:], preferred_element_type=jnp.float32)
    out_ref[:, :_TOKEN_SIZE] = e + eb_ref[:]
    out_ref[:, _TOKEN_SIZE:] = tok_ref[:] + pe_ref[:]


@functools.partial(jax.jit, static_argnames=())
def kernel(tokens, energies, W, b, emb):
    batch = tokens.shape[0]
    n_tok, tsz = emb.shape[0], emb.shape[1]
    flat_in = (n_tok - 1) * tsz
    flat_out = n_tok * tsz
    tokens2d = tokens.reshape(batch, flat_in)
    pe_row = emb[1:].reshape(1, flat_in)
    e_bias = (b + emb[0]).reshape(1, tsz)

    grid = (batch // _BB,)
    out2d = pl.pallas_call(
        _body,
        grid=grid,
        in_specs=[
            pl.BlockSpec((_BB, flat_in), lambda i: (i, 0)),
            pl.BlockSpec((_BB, tsz), lambda i: (i, 0)),
            pl.BlockSpec((tsz, tsz), lambda i: (0, 0)),
            pl.BlockSpec((1, tsz), lambda i: (0, 0)),
            pl.BlockSpec((1, flat_in), lambda i: (0, 0)),
        ],
        out_specs=pl.BlockSpec((_BB, flat_out), lambda i: (i, 0)),
        out_shape=jax.ShapeDtypeStruct((batch, flat_out), jnp.float32),
    )(tokens2d, energies, W, e_bias, pe_row)
    return out2d.reshape(batch, n_tok, tsz)
